# Initial kernel scaffold; baseline (speedup 1.0000x reference)
import jax
import jax.numpy as jnp
from jax.experimental import pallas as pl


def kernel(atomic_charges, distances, idx_i, idx_j):
    raise NotImplementedError("write your pallas kernel here")


# speedup vs baseline: 173.2457x; 173.2457x over previous
import functools

import jax
import jax.numpy as jnp
from jax import lax
from jax.experimental import pallas as pl
from jax.experimental.pallas import tpu as pltpu
from jax.experimental.pallas import tpu_sc as plsc

N_NODES = 50000
N_EDGES = 3200000
CUTOFF = 10.0
CUTOFF_SR = 5.0
KEHALF = 7.199822675975274

NC = 2
NS = 16
NW = NC * NS
LANES = 16

E_W = N_EDGES // NW
CHUNK = 4000
N_CHUNKS = E_W // CHUNK
VECS = CHUNK // LANES

SLICE = 3200
N_PAD = NS * SLICE


def _rsqrt(s):
    xi = plsc.bitcast(s, jnp.int32)
    yi = jnp.int32(0x5F3759DF) - lax.shift_right_logical(xi, jnp.int32(1))
    y = plsc.bitcast(yi, jnp.float32)
    half_s = 0.5 * s
    for _ in range(3):
        y = y * (1.5 - half_s * y * y)
    return y


def _sc_body(q_hbm, d_hbm, ii_hbm, jj_hbm, out_hbm,
             q_v, acc_v, d_v, ii_v, jj_v):
    cid = lax.axis_index("c")
    sid = lax.axis_index("s")
    wid = sid * NC + cid
    ebase = wid * E_W

    inv_c2 = 1.0 / (CUTOFF * CUTOFF)
    two_over_c = 2.0 / CUTOFF
    inv_sr = 1.0 / CUTOFF_SR

    pltpu.sync_copy(q_hbm, q_v)

    zeros = jnp.zeros((LANES,), jnp.float32)

    def zero_body(v, _):
        acc_v[pl.ds(v * LANES, LANES)] = zeros
        return _

    lax.fori_loop(0, N_PAD // LANES, zero_body, 0, unroll=4)

    def chunk_body(k, _):
        base = ebase + k * CHUNK
        pltpu.sync_copy(d_hbm.at[pl.ds(base, CHUNK)], d_v)
        pltpu.sync_copy(ii_hbm.at[pl.ds(base, CHUNK)], ii_v)
        pltpu.sync_copy(jj_hbm.at[pl.ds(base, CHUNK)], jj_v)

        def vec_body(v, _):
            off = v * LANES
            ii = ii_v[pl.ds(off, LANES)]
            jj = jj_v[pl.ds(off, LANES)]
            d = d_v[pl.ds(off, LANES)]
            qi = plsc.load_gather(q_v, [ii])
            qj = plsc.load_gather(q_v, [jj])

            s = d * d + 1.0
            inv_ds = _rsqrt(s)
            ds = s * inv_ds
            inv_d = 1.0 / d

            x = d * inv_sr
            x2 = x * x
            x3 = x2 * x
            fx = 1.0 + x3 * (-10.0 + x * (15.0 - 6.0 * x))
            sw_off = jnp.where(d < CUTOFF_SR, fx, 0.0)
            sw_on = 1.0 - sw_off

            e_ord = inv_d + d * inv_c2 - two_over_c
            e_sh = inv_ds + ds * inv_c2 - two_over_c
            e = (KEHALF * qi) * qj * (sw_off * e_sh + sw_on * e_ord)
            e = jnp.where(d <= CUTOFF, e, 0.0)
            plsc.addupdate_scatter(acc_v, [ii], e)
            return _

        lax.fori_loop(0, VECS, vec_body, 0)
        return _

    lax.fori_loop(0, N_CHUNKS, chunk_body, 0)

    pltpu.sync_copy(acc_v, out_hbm.at[pl.ds(wid * N_PAD, N_PAD)])


def _combine_body(parts_ref, o_ref):
    o_ref[...] = jnp.sum(parts_ref[...], axis=0)


@jax.jit
def kernel(atomic_charges, distances, idx_i, idx_j):
    mesh = plsc.VectorSubcoreMesh(core_axis_name="c", subcore_axis_name="s")
    sc_fn = pl.kernel(
        _sc_body,
        out_type=jax.ShapeDtypeStruct((NW * N_PAD,), jnp.float32),
        mesh=mesh,
        compiler_params=pltpu.CompilerParams(needs_layout_passes=False),
        scratch_types=[
            pltpu.VMEM((N_NODES,), jnp.float32),
            pltpu.VMEM((N_PAD,), jnp.float32),
            pltpu.VMEM((CHUNK,), jnp.float32),
            pltpu.VMEM((CHUNK,), jnp.int32),
            pltpu.VMEM((CHUNK,), jnp.int32),
        ],
    )
    parts = sc_fn(atomic_charges, distances, idx_i, idx_j)

    combined = pl.pallas_call(
        _combine_body,
        out_shape=jax.ShapeDtypeStruct((N_PAD // 128, 128), jnp.float32),
    )(parts.reshape(NW, N_PAD // 128, 128))
    return combined.reshape(-1)[:N_NODES]


# speedup vs baseline: 317.5687x; 1.8331x over previous
import jax
import jax.numpy as jnp
from jax import lax
from jax.experimental import pallas as pl
from jax.experimental.pallas import tpu as pltpu
from jax.experimental.pallas import tpu_sc as plsc

N_NODES = 50000
N_EDGES = 3200000
CUTOFF = 10.0
CUTOFF_SR = 5.0
KEHALF = 7.199822675975274

NC = 2
NS = 16
NW = NC * NS
LANES = 16

E_W = N_EDGES // NW
CHUNK = 4000
N_CHUNKS = E_W // CHUNK
VECS = CHUNK // LANES

SLICE = 3200
N_PAD = NS * SLICE


def _rsqrt(s, iters):
    xi = plsc.bitcast(s, jnp.int32)
    yi = jnp.int32(0x5F3759DF) - lax.shift_right_logical(xi, jnp.int32(1))
    y = plsc.bitcast(yi, jnp.float32)
    half_s = 0.5 * s
    for _ in range(iters):
        y = y * (1.5 - half_s * y * y)
    return y


def _sc_body(q_hbm, d_hbm, ii_hbm, jj_hbm, out_hbm,
             q_v, acc_v, d_v0, ii_v0, jj_v0, d_v1, ii_v1, jj_v1,
             sem0, sem1):
    cid = lax.axis_index("c")
    sid = lax.axis_index("s")
    wid = sid * NC + cid
    ebase = wid * E_W

    inv_c2 = 1.0 / (CUTOFF * CUTOFF)
    two_over_c = 2.0 / CUTOFF
    inv_sr = 1.0 / CUTOFF_SR

    bufs = ((d_v0, ii_v0, jj_v0, sem0), (d_v1, ii_v1, jj_v1, sem1))

    def start(k, p):
        d_b, ii_b, jj_b, sem = bufs[p]
        base = ebase + k * CHUNK
        pltpu.async_copy(d_hbm.at[pl.ds(base, CHUNK)], d_b, sem)
        pltpu.async_copy(ii_hbm.at[pl.ds(base, CHUNK)], ii_b, sem)
        pltpu.async_copy(jj_hbm.at[pl.ds(base, CHUNK)], jj_b, sem)

    def wait(k, p):
        d_b, ii_b, jj_b, sem = bufs[p]
        base = ebase + k * CHUNK
        pltpu.make_async_copy(d_hbm.at[pl.ds(base, CHUNK)], d_b, sem).wait()
        pltpu.make_async_copy(ii_hbm.at[pl.ds(base, CHUNK)], ii_b, sem).wait()
        pltpu.make_async_copy(jj_hbm.at[pl.ds(base, CHUNK)], jj_b, sem).wait()

    pltpu.sync_copy(q_hbm, q_v)

    zeros = jnp.zeros((LANES,), jnp.float32)

    def zero_body(v, _):
        acc_v[pl.ds(v * LANES, LANES)] = zeros
        return _

    lax.fori_loop(0, N_PAD // LANES, zero_body, 0, unroll=4)

    start(0, 0)
    lane_off = lax.iota(jnp.int32, LANES) * VECS

    def process(p):
        d_b, ii_b, jj_b, _ = bufs[p]

        def vec_body(v, _):
            iv = lane_off + v
            ii = plsc.load_gather(ii_b, [iv])
            jj = plsc.load_gather(jj_b, [iv])
            d = plsc.load_gather(d_b, [iv])
            qi = plsc.load_gather(q_v, [ii])
            qj = plsc.load_gather(q_v, [jj])

            d2 = d * d
            s = d2 + 1.0
            inv_ds = _rsqrt(s, 2)
            inv_d = _rsqrt(d2, 2)
            ds = s * inv_ds

            x = d * inv_sr
            x2 = x * x
            x3 = x2 * x
            fx = 1.0 + x3 * (-10.0 + x * (15.0 - 6.0 * x))
            sw_off = jnp.where(d < CUTOFF_SR, fx, 0.0)

            e_ord = inv_d + d * inv_c2 - two_over_c
            e_sh = inv_ds + ds * inv_c2 - two_over_c
            e = (KEHALF * qi) * qj * (e_ord + sw_off * (e_sh - e_ord))
            e = jnp.where(d <= CUTOFF, e, 0.0)
            plsc.addupdate_scatter(acc_v, [ii], e)
            return _

        lax.fori_loop(0, VECS, vec_body, 0, unroll=2)

    def chunk_body(k, _):
        for p in range(2):
            kk = 2 * k + p

            @pl.when(kk + 1 < N_CHUNKS)
            def _start():
                start(kk + 1, 1 - p)

            wait(kk, p)
            process(p)
        return _

    lax.fori_loop(0, N_CHUNKS // 2, chunk_body, 0)
    wait(N_CHUNKS - 1, (N_CHUNKS - 1) % 2)
    process((N_CHUNKS - 1) % 2)

    pltpu.sync_copy(acc_v, out_hbm.at[pl.ds(wid * N_PAD, N_PAD)])


def _combine_body(parts_ref, o_ref):
    o_ref[...] = jnp.sum(parts_ref[...], axis=0)


@jax.jit
def kernel(atomic_charges, distances, idx_i, idx_j):
    mesh = plsc.VectorSubcoreMesh(core_axis_name="c", subcore_axis_name="s")
    sc_fn = pl.kernel(
        _sc_body,
        out_type=jax.ShapeDtypeStruct((NW * N_PAD,), jnp.float32),
        mesh=mesh,
        compiler_params=pltpu.CompilerParams(needs_layout_passes=False),
        scratch_types=[
            pltpu.VMEM((N_NODES,), jnp.float32),
            pltpu.VMEM((N_PAD,), jnp.float32),
            pltpu.VMEM((CHUNK,), jnp.float32),
            pltpu.VMEM((CHUNK,), jnp.int32),
            pltpu.VMEM((CHUNK,), jnp.int32),
            pltpu.VMEM((CHUNK,), jnp.float32),
            pltpu.VMEM((CHUNK,), jnp.int32),
            pltpu.VMEM((CHUNK,), jnp.int32),
            pltpu.SemaphoreType.DMA,
            pltpu.SemaphoreType.DMA,
        ],
    )
    parts = sc_fn(atomic_charges, distances, idx_i, idx_j)

    combined = pl.pallas_call(
        _combine_body,
        out_shape=jax.ShapeDtypeStruct((N_PAD // 128, 128), jnp.float32),
    )(parts.reshape(NW, N_PAD // 128, 128))
    return combined.reshape(-1)[:N_NODES]


# speedup vs baseline: 621.7855x; 1.9580x over previous
import jax
import jax.numpy as jnp
from jax import lax
from jax.experimental import pallas as pl
from jax.experimental.pallas import tpu as pltpu
from jax.experimental.pallas import tpu_sc as plsc

N_NODES = 50000
N_EDGES = 3200000
CUTOFF = 10.0
CUTOFF_SR = 5.0
KEHALF = 7.199822675975274

NC = 2
NS = 16
NW = NC * NS
LANES = 16

E_W = N_EDGES // NW
CHUNK = 4000
N_CHUNKS = E_W // CHUNK
VECS = CHUNK // LANES

SLICE = 3200
N_PAD = NS * SLICE


def _rsqrt(s, iters):
    xi = plsc.bitcast(s, jnp.int32)
    yi = jnp.int32(0x5F3759DF) - lax.shift_right_logical(xi, jnp.int32(1))
    y = plsc.bitcast(yi, jnp.float32)
    half_s = 0.5 * s
    for _ in range(iters):
        y = y * (1.5 - half_s * y * y)
    return y


def _sc_body(q_hbm, d_hbm, ii_hbm, jj_hbm, out_hbm,
             q_v, acc_v, d_v0, ii_v0, jj_v0, d_v1, ii_v1, jj_v1,
             sem0, sem1):
    cid = lax.axis_index("c")
    sid = lax.axis_index("s")
    wid = sid * NC + cid
    ebase = wid * E_W

    inv_c2 = 1.0 / (CUTOFF * CUTOFF)
    two_over_c = 2.0 / CUTOFF
    inv_sr = 1.0 / CUTOFF_SR

    bufs = ((d_v0, ii_v0, jj_v0, sem0), (d_v1, ii_v1, jj_v1, sem1))

    def start(k, p):
        d_b, ii_b, jj_b, sem = bufs[p]
        base = ebase + k * CHUNK
        pltpu.async_copy(d_hbm.at[pl.ds(base, CHUNK)], d_b, sem)
        pltpu.async_copy(ii_hbm.at[pl.ds(base, CHUNK)], ii_b, sem)
        pltpu.async_copy(jj_hbm.at[pl.ds(base, CHUNK)], jj_b, sem)

    def wait(k, p):
        d_b, ii_b, jj_b, sem = bufs[p]
        base = ebase + k * CHUNK
        pltpu.make_async_copy(d_hbm.at[pl.ds(base, CHUNK)], d_b, sem).wait()
        pltpu.make_async_copy(ii_hbm.at[pl.ds(base, CHUNK)], ii_b, sem).wait()
        pltpu.make_async_copy(jj_hbm.at[pl.ds(base, CHUNK)], jj_b, sem).wait()

    pltpu.sync_copy(q_hbm, q_v)

    zeros = jnp.zeros((LANES,), jnp.float32)

    def zero_body(v, _):
        acc_v[pl.ds(v * LANES, LANES)] = zeros
        return _

    lax.fori_loop(0, N_PAD // LANES, zero_body, 0, unroll=4)

    start(0, 0)
    lane_off = lax.iota(jnp.int32, LANES) * VECS

    def process(p):
        d_b, ii_b, jj_b, _ = bufs[p]

        @plsc.parallel_loop(0, VECS, step=1, unroll=4)
        def vec_body(v):
            iv = lane_off + v
            ii = plsc.load_gather(ii_b, [iv])
            jj = plsc.load_gather(jj_b, [iv])
            d = plsc.load_gather(d_b, [iv])
            qi = plsc.load_gather(q_v, [ii])
            qj = plsc.load_gather(q_v, [jj])

            d2 = d * d
            s = d2 + 1.0
            inv_ds = _rsqrt(s, 2)
            inv_d = _rsqrt(d2, 2)
            ds = s * inv_ds

            x = d * inv_sr
            x2 = x * x
            x3 = x2 * x
            fx = 1.0 + x3 * (-10.0 + x * (15.0 - 6.0 * x))
            sw_off = jnp.where(d < CUTOFF_SR, fx, 0.0)

            e_ord = inv_d + d * inv_c2 - two_over_c
            e_sh = inv_ds + ds * inv_c2 - two_over_c
            e = (KEHALF * qi) * qj * (e_ord + sw_off * (e_sh - e_ord))
            e = jnp.where(d <= CUTOFF, e, 0.0)
            plsc.addupdate_scatter(acc_v, [ii], e)

    def chunk_body(k, _):
        for p in range(2):
            kk = 2 * k + p

            @pl.when(kk + 1 < N_CHUNKS)
            def _start():
                start(kk + 1, 1 - p)

            wait(kk, p)
            process(p)
        return _

    lax.fori_loop(0, N_CHUNKS // 2, chunk_body, 0)
    wait(N_CHUNKS - 1, (N_CHUNKS - 1) % 2)
    process((N_CHUNKS - 1) % 2)

    pltpu.sync_copy(acc_v, out_hbm.at[pl.ds(wid * N_PAD, N_PAD)])


def _combine_body(parts_ref, o_ref):
    o_ref[...] = jnp.sum(parts_ref[...], axis=0)


@jax.jit
def kernel(atomic_charges, distances, idx_i, idx_j):
    mesh = plsc.VectorSubcoreMesh(core_axis_name="c", subcore_axis_name="s")
    sc_fn = pl.kernel(
        _sc_body,
        out_type=jax.ShapeDtypeStruct((NW * N_PAD,), jnp.float32),
        mesh=mesh,
        compiler_params=pltpu.CompilerParams(needs_layout_passes=False),
        scratch_types=[
            pltpu.VMEM((N_NODES,), jnp.float32),
            pltpu.VMEM((N_PAD,), jnp.float32),
            pltpu.VMEM((CHUNK,), jnp.float32),
            pltpu.VMEM((CHUNK,), jnp.int32),
            pltpu.VMEM((CHUNK,), jnp.int32),
            pltpu.VMEM((CHUNK,), jnp.float32),
            pltpu.VMEM((CHUNK,), jnp.int32),
            pltpu.VMEM((CHUNK,), jnp.int32),
            pltpu.SemaphoreType.DMA,
            pltpu.SemaphoreType.DMA,
        ],
    )
    parts = sc_fn(atomic_charges, distances, idx_i, idx_j)

    combined = pl.pallas_call(
        _combine_body,
        out_shape=jax.ShapeDtypeStruct((N_PAD // 128, 128), jnp.float32),
    )(parts.reshape(NW, N_PAD // 128, 128))
    return combined.reshape(-1)[:N_NODES]


# speedup vs baseline: 629.3299x; 1.0121x over previous
import jax
import jax.numpy as jnp
from jax import lax
from jax.experimental import pallas as pl
from jax.experimental.pallas import tpu as pltpu
from jax.experimental.pallas import tpu_sc as plsc

N_NODES = 50000
N_EDGES = 3200000
CUTOFF = 10.0
CUTOFF_SR = 5.0
KEHALF = 7.199822675975274

NC = 2
NS = 16
NW = NC * NS
LANES = 16

E_W = N_EDGES // NW
CHUNK = 4000
N_CHUNKS = E_W // CHUNK
VECS = CHUNK // LANES

SLICE = 3200
N_PAD = NS * SLICE


def _rsqrt(s, iters):
    xi = plsc.bitcast(s, jnp.int32)
    yi = jnp.int32(0x5F3759DF) - lax.shift_right_logical(xi, jnp.int32(1))
    y = plsc.bitcast(yi, jnp.float32)
    half_s = 0.5 * s
    for _ in range(iters):
        y = y * (1.5 - half_s * y * y)
    return y


def _sc_body(q_hbm, d_hbm, ii_hbm, jj_hbm, out_hbm,
             q_v, acc_v, d_v0, ii_v0, jj_v0, d_v1, ii_v1, jj_v1,
             sem0, sem1):
    cid = lax.axis_index("c")
    sid = lax.axis_index("s")
    wid = sid * NC + cid
    ebase = wid * E_W

    inv_c2 = 1.0 / (CUTOFF * CUTOFF)
    two_over_c = 2.0 / CUTOFF
    inv_sr = 1.0 / CUTOFF_SR

    bufs = ((d_v0, ii_v0, jj_v0, sem0), (d_v1, ii_v1, jj_v1, sem1))

    def start(k, p):
        d_b, ii_b, jj_b, sem = bufs[p]
        base = ebase + k * CHUNK
        pltpu.async_copy(d_hbm.at[pl.ds(base, CHUNK)], d_b, sem)
        pltpu.async_copy(ii_hbm.at[pl.ds(base, CHUNK)], ii_b, sem)
        pltpu.async_copy(jj_hbm.at[pl.ds(base, CHUNK)], jj_b, sem)

    def wait(k, p):
        d_b, ii_b, jj_b, sem = bufs[p]
        base = ebase + k * CHUNK
        pltpu.make_async_copy(d_hbm.at[pl.ds(base, CHUNK)], d_b, sem).wait()
        pltpu.make_async_copy(ii_hbm.at[pl.ds(base, CHUNK)], ii_b, sem).wait()
        pltpu.make_async_copy(jj_hbm.at[pl.ds(base, CHUNK)], jj_b, sem).wait()

    pltpu.sync_copy(q_hbm, q_v)

    zeros = jnp.zeros((LANES,), jnp.float32)

    def zero_body(v, _):
        acc_v[pl.ds(v * LANES, LANES)] = zeros
        return _

    lax.fori_loop(0, N_PAD // LANES, zero_body, 0, unroll=4)

    start(0, 0)
    lane_off = lax.iota(jnp.int32, LANES) * VECS

    def process(p):
        d_b, ii_b, jj_b, _ = bufs[p]

        @plsc.parallel_loop(0, VECS, step=1, unroll=10)
        def vec_body(v):
            iv = lane_off + v
            ii = plsc.load_gather(ii_b, [iv])
            jj = plsc.load_gather(jj_b, [iv])
            d = plsc.load_gather(d_b, [iv])
            qi = plsc.load_gather(q_v, [ii])
            qj = plsc.load_gather(q_v, [jj])

            d2 = d * d
            s = d2 + 1.0
            inv_ds = _rsqrt(s, 1)
            inv_d = _rsqrt(d2, 1)
            ds = s * inv_ds

            x = d * inv_sr
            x2 = x * x
            x3 = x2 * x
            fx = 1.0 + x3 * (-10.0 + x * (15.0 - 6.0 * x))
            sw_off = jnp.where(d < CUTOFF_SR, fx, 0.0)

            e_ord = inv_d + d * inv_c2 - two_over_c
            e_sh = inv_ds + ds * inv_c2 - two_over_c
            e = (KEHALF * qi) * qj * (e_ord + sw_off * (e_sh - e_ord))
            e = jnp.where(d <= CUTOFF, e, 0.0)
            plsc.addupdate_scatter(acc_v, [ii], e)

    def chunk_body(k, _):
        for p in range(2):
            kk = 2 * k + p

            @pl.when(kk + 1 < N_CHUNKS)
            def _start():
                start(kk + 1, 1 - p)

            wait(kk, p)
            process(p)
        return _

    lax.fori_loop(0, N_CHUNKS // 2, chunk_body, 0)
    wait(N_CHUNKS - 1, (N_CHUNKS - 1) % 2)
    process((N_CHUNKS - 1) % 2)

    pltpu.sync_copy(acc_v, out_hbm.at[pl.ds(wid * N_PAD, N_PAD)])


def _combine_body(parts_ref, o_ref):
    o_ref[...] = jnp.sum(parts_ref[...], axis=0)


@jax.jit
def kernel(atomic_charges, distances, idx_i, idx_j):
    mesh = plsc.VectorSubcoreMesh(core_axis_name="c", subcore_axis_name="s")
    sc_fn = pl.kernel(
        _sc_body,
        out_type=jax.ShapeDtypeStruct((NW * N_PAD,), jnp.float32),
        mesh=mesh,
        compiler_params=pltpu.CompilerParams(needs_layout_passes=False),
        scratch_types=[
            pltpu.VMEM((N_NODES,), jnp.float32),
            pltpu.VMEM((N_PAD,), jnp.float32),
            pltpu.VMEM((CHUNK,), jnp.float32),
            pltpu.VMEM((CHUNK,), jnp.int32),
            pltpu.VMEM((CHUNK,), jnp.int32),
            pltpu.VMEM((CHUNK,), jnp.float32),
            pltpu.VMEM((CHUNK,), jnp.int32),
            pltpu.VMEM((CHUNK,), jnp.int32),
            pltpu.SemaphoreType.DMA,
            pltpu.SemaphoreType.DMA,
        ],
    )
    parts = sc_fn(atomic_charges, distances, idx_i, idx_j)

    combined = pl.pallas_call(
        _combine_body,
        out_shape=jax.ShapeDtypeStruct((N_PAD // 128, 128), jnp.float32),
    )(parts.reshape(NW, N_PAD // 128, 128))
    return combined.reshape(-1)[:N_NODES]


# speedup vs baseline: 636.8122x; 1.0119x over previous
import jax
import jax.numpy as jnp
from jax import lax
from jax.experimental import pallas as pl
from jax.experimental.pallas import tpu as pltpu
from jax.experimental.pallas import tpu_sc as plsc

N_NODES = 50000
N_EDGES = 3200000
CUTOFF = 10.0
CUTOFF_SR = 5.0
KEHALF = 7.199822675975274

NC = 2
NS = 16
NW = NC * NS
LANES = 16

E_W = N_EDGES // NW
CHUNK = 4000
N_CHUNKS = E_W // CHUNK
VECS = CHUNK // LANES

SLICE = 3200
N_PAD = NS * SLICE


def _rsqrt(s, iters):
    xi = plsc.bitcast(s, jnp.int32)
    yi = jnp.int32(0x5F3759DF) - lax.shift_right_logical(xi, jnp.int32(1))
    y = plsc.bitcast(yi, jnp.float32)
    half_s = 0.5 * s
    for _ in range(iters):
        y = y * (1.5 - half_s * y * y)
    return y


def _sc_body(q_hbm, d_hbm, ii_hbm, jj_hbm, out_hbm,
             q_v, acc_v, d_v0, ii_v0, jj_v0, d_v1, ii_v1, jj_v1,
             sem0, sem1):
    cid = lax.axis_index("c")
    sid = lax.axis_index("s")
    wid = sid * NC + cid
    ebase = wid * E_W

    inv_c2 = 1.0 / (CUTOFF * CUTOFF)
    two_over_c = 2.0 / CUTOFF
    PA = -10.0 / CUTOFF_SR**3
    PB = 15.0 / CUTOFF_SR**4
    PC = -6.0 / CUTOFF_SR**5

    bufs = ((d_v0, ii_v0, jj_v0, sem0), (d_v1, ii_v1, jj_v1, sem1))

    def start(k, p):
        d_b, ii_b, jj_b, sem = bufs[p]
        base = ebase + k * CHUNK
        pltpu.async_copy(d_hbm.at[pl.ds(base, CHUNK)], d_b, sem)
        pltpu.async_copy(ii_hbm.at[pl.ds(base, CHUNK)], ii_b, sem)
        pltpu.async_copy(jj_hbm.at[pl.ds(base, CHUNK)], jj_b, sem)

    def wait(k, p):
        d_b, ii_b, jj_b, sem = bufs[p]
        base = ebase + k * CHUNK
        pltpu.make_async_copy(d_hbm.at[pl.ds(base, CHUNK)], d_b, sem).wait()
        pltpu.make_async_copy(ii_hbm.at[pl.ds(base, CHUNK)], ii_b, sem).wait()
        pltpu.make_async_copy(jj_hbm.at[pl.ds(base, CHUNK)], jj_b, sem).wait()

    pltpu.sync_copy(q_hbm, q_v)

    zeros = jnp.zeros((LANES,), jnp.float32)

    def zero_body(v, _):
        acc_v[pl.ds(v * LANES, LANES)] = zeros
        return _

    lax.fori_loop(0, N_PAD // LANES, zero_body, 0, unroll=4)

    start(0, 0)
    lane_off = lax.iota(jnp.int32, LANES) * VECS

    def process(p):
        d_b, ii_b, jj_b, _ = bufs[p]

        @plsc.parallel_loop(0, VECS, step=1, unroll=10)
        def vec_body(v):
            iv = lane_off + v
            ii = plsc.load_gather(ii_b, [iv])
            jj = plsc.load_gather(jj_b, [iv])
            d = plsc.load_gather(d_b, [iv])
            qi = plsc.load_gather(q_v, [ii])
            qj = plsc.load_gather(q_v, [jj])

            d2 = d * d
            s = d2 + 1.0
            m = d2 * s
            r = _rsqrt(m, 2)
            inv_ds = d * r
            ds = s * inv_ds
            inv_d = ds * r

            d3 = d2 * d
            p = PA + d * (PB + PC * d)
            fx = 1.0 + d3 * p
            sw_off = jnp.where(d < CUTOFF_SR, fx, 0.0)

            e_ord = inv_d + d * inv_c2 - two_over_c
            diff = (inv_ds - inv_d) + (ds - d) * inv_c2
            e = (KEHALF * qi) * qj * (e_ord + sw_off * diff)
            e = jnp.where(d <= CUTOFF, e, 0.0)
            plsc.addupdate_scatter(acc_v, [ii], e)

    def chunk_body(k, _):
        for p in range(2):
            kk = 2 * k + p

            @pl.when(kk + 1 < N_CHUNKS)
            def _start():
                start(kk + 1, 1 - p)

            wait(kk, p)
            process(p)
        return _

    lax.fori_loop(0, N_CHUNKS // 2, chunk_body, 0)
    wait(N_CHUNKS - 1, (N_CHUNKS - 1) % 2)
    process((N_CHUNKS - 1) % 2)

    pltpu.sync_copy(acc_v, out_hbm.at[pl.ds(wid * N_PAD, N_PAD)])


def _combine_body(parts_ref, o_ref):
    o_ref[...] = jnp.sum(parts_ref[...], axis=0)


@jax.jit
def kernel(atomic_charges, distances, idx_i, idx_j):
    mesh = plsc.VectorSubcoreMesh(core_axis_name="c", subcore_axis_name="s")
    sc_fn = pl.kernel(
        _sc_body,
        out_type=jax.ShapeDtypeStruct((NW * N_PAD,), jnp.float32),
        mesh=mesh,
        compiler_params=pltpu.CompilerParams(needs_layout_passes=False),
        scratch_types=[
            pltpu.VMEM((N_NODES,), jnp.float32),
            pltpu.VMEM((N_PAD,), jnp.float32),
            pltpu.VMEM((CHUNK,), jnp.float32),
            pltpu.VMEM((CHUNK,), jnp.int32),
            pltpu.VMEM((CHUNK,), jnp.int32),
            pltpu.VMEM((CHUNK,), jnp.float32),
            pltpu.VMEM((CHUNK,), jnp.int32),
            pltpu.VMEM((CHUNK,), jnp.int32),
            pltpu.SemaphoreType.DMA,
            pltpu.SemaphoreType.DMA,
        ],
    )
    parts = sc_fn(atomic_charges, distances, idx_i, idx_j)

    combined = pl.pallas_call(
        _combine_body,
        out_shape=jax.ShapeDtypeStruct((N_PAD // 128, 128), jnp.float32),
    )(parts.reshape(NW, N_PAD // 128, 128))
    return combined.reshape(-1)[:N_NODES]


# speedup vs baseline: 637.0241x; 1.0003x over previous
import jax
import jax.numpy as jnp
from jax import lax
from jax.experimental import pallas as pl
from jax.experimental.pallas import tpu as pltpu
from jax.experimental.pallas import tpu_sc as plsc

N_NODES = 50000
N_EDGES = 3200000
CUTOFF = 10.0
CUTOFF_SR = 5.0
KEHALF = 7.199822675975274

NC = 2
NS = 16
NW = NC * NS
LANES = 16

E_W = N_EDGES // NW
CHUNK = 4000
N_CHUNKS = E_W // CHUNK
VECS = CHUNK // LANES

SLICE = 3200
N_PAD = NS * SLICE


def _rsqrt(s, iters):
    xi = plsc.bitcast(s, jnp.int32)
    yi = jnp.int32(0x5F3759DF) - lax.shift_right_logical(xi, jnp.int32(1))
    y = plsc.bitcast(yi, jnp.float32)
    half_s = 0.5 * s
    for _ in range(iters):
        y = y * (1.5 - half_s * y * y)
    return y


def _sc_body(q_hbm, d_hbm, ii_hbm, jj_hbm, out_hbm,
             q_v, acc_v, d_v0, ii_v0, jj_v0, d_v1, ii_v1, jj_v1,
             sem0, sem1):
    cid = lax.axis_index("c")
    sid = lax.axis_index("s")
    wid = sid * NC + cid
    ebase = wid * E_W

    inv_c2 = 1.0 / (CUTOFF * CUTOFF)
    two_over_c = 2.0 / CUTOFF
    PA = -10.0 / CUTOFF_SR**3
    PB = 15.0 / CUTOFF_SR**4
    PC = -6.0 / CUTOFF_SR**5

    bufs = ((d_v0, ii_v0, jj_v0, sem0), (d_v1, ii_v1, jj_v1, sem1))

    def start(k, p):
        d_b, ii_b, jj_b, sem = bufs[p]
        base = ebase + k * CHUNK
        pltpu.async_copy(d_hbm.at[pl.ds(base, CHUNK)], d_b, sem)
        pltpu.async_copy(ii_hbm.at[pl.ds(base, CHUNK)], ii_b, sem)
        pltpu.async_copy(jj_hbm.at[pl.ds(base, CHUNK)], jj_b, sem)

    def wait(k, p):
        d_b, ii_b, jj_b, sem = bufs[p]
        base = ebase + k * CHUNK
        pltpu.make_async_copy(d_hbm.at[pl.ds(base, CHUNK)], d_b, sem).wait()
        pltpu.make_async_copy(ii_hbm.at[pl.ds(base, CHUNK)], ii_b, sem).wait()
        pltpu.make_async_copy(jj_hbm.at[pl.ds(base, CHUNK)], jj_b, sem).wait()

    with jax.named_scope("qcopy"):
        pltpu.sync_copy(q_hbm, q_v)

    zeros = jnp.zeros((LANES,), jnp.float32)

    def zero_body(v, _):
        acc_v[pl.ds(v * LANES, LANES)] = zeros
        return _

    with jax.named_scope("zero"):
        lax.fori_loop(0, N_PAD // LANES, zero_body, 0, unroll=4)

    start(0, 0)
    lane_off = lax.iota(jnp.int32, LANES) * VECS

    def process(p):
        d_b, ii_b, jj_b, _ = bufs[p]

        @plsc.parallel_loop(0, VECS, step=1, unroll=10)
        def vec_body(v):
            iv = lane_off + v
            ii = plsc.load_gather(ii_b, [iv])
            jj = plsc.load_gather(jj_b, [iv])
            d = plsc.load_gather(d_b, [iv])
            qi = plsc.load_gather(q_v, [ii])
            qj = plsc.load_gather(q_v, [jj])

            d2 = d * d
            s = d2 + 1.0
            m = d2 * s
            r = _rsqrt(m, 2)
            inv_ds = d * r
            ds = s * inv_ds
            inv_d = ds * r

            d3 = d2 * d
            p = PA + d * (PB + PC * d)
            fx = 1.0 + d3 * p
            sw_off = jnp.where(d < CUTOFF_SR, fx, 0.0)

            e_ord = inv_d + d * inv_c2 - two_over_c
            diff = (inv_ds - inv_d) + (ds - d) * inv_c2
            e = (KEHALF * qi) * qj * (e_ord + sw_off * diff)
            e = jnp.where(d <= CUTOFF, e, 0.0)
            plsc.addupdate_scatter(acc_v, [ii], e)

    def chunk_body(k, _):
        for p in range(2):
            kk = 2 * k + p

            @pl.when(kk + 1 < N_CHUNKS)
            def _start():
                start(kk + 1, 1 - p)

            wait(kk, p)
            process(p)
        return _

    with jax.named_scope("main"):
        lax.fori_loop(0, N_CHUNKS // 2, chunk_body, 0)
        wait(N_CHUNKS - 1, (N_CHUNKS - 1) % 2)
        process((N_CHUNKS - 1) % 2)

    with jax.named_scope("wout"):
        pltpu.sync_copy(acc_v, out_hbm.at[pl.ds(wid * N_PAD, N_PAD)])


def _combine_body(parts_ref, o_ref):
    o_ref[...] = jnp.sum(parts_ref[...], axis=0)


@jax.jit
def kernel(atomic_charges, distances, idx_i, idx_j):
    mesh = plsc.VectorSubcoreMesh(core_axis_name="c", subcore_axis_name="s")
    sc_fn = pl.kernel(
        _sc_body,
        out_type=jax.ShapeDtypeStruct((NW * N_PAD,), jnp.float32),
        mesh=mesh,
        compiler_params=pltpu.CompilerParams(needs_layout_passes=False),
        scratch_types=[
            pltpu.VMEM((N_NODES,), jnp.float32),
            pltpu.VMEM((N_PAD,), jnp.float32),
            pltpu.VMEM((CHUNK,), jnp.float32),
            pltpu.VMEM((CHUNK,), jnp.int32),
            pltpu.VMEM((CHUNK,), jnp.int32),
            pltpu.VMEM((CHUNK,), jnp.float32),
            pltpu.VMEM((CHUNK,), jnp.int32),
            pltpu.VMEM((CHUNK,), jnp.int32),
            pltpu.SemaphoreType.DMA,
            pltpu.SemaphoreType.DMA,
        ],
    )
    parts = sc_fn(atomic_charges, distances, idx_i, idx_j)

    combined = pl.pallas_call(
        _combine_body,
        out_shape=jax.ShapeDtypeStruct((N_PAD // 128, 128), jnp.float32),
    )(parts.reshape(NW, N_PAD // 128, 128))
    return combined.reshape(-1)[:N_NODES]


# speedup vs baseline: 646.9260x; 1.0155x over previous
import jax
import jax.numpy as jnp
from jax import lax
from jax.experimental import pallas as pl
from jax.experimental.pallas import tpu as pltpu
from jax.experimental.pallas import tpu_sc as plsc

N_NODES = 50000
N_EDGES = 3200000
CUTOFF = 10.0
CUTOFF_SR = 5.0
KEHALF = 7.199822675975274

NC = 2
NS = 16
NW = NC * NS
LANES = 16

E_W = N_EDGES // NW
CHUNK = 4000
N_CHUNKS = E_W // CHUNK
VECS = CHUNK // LANES

SLICE = 3200
N_PAD = NS * SLICE


def _rsqrt(s, iters):
    xi = plsc.bitcast(s, jnp.int32)
    yi = jnp.int32(0x5F3759DF) - lax.shift_right_logical(xi, jnp.int32(1))
    y = plsc.bitcast(yi, jnp.float32)
    half_s = 0.5 * s
    for _ in range(iters):
        y = y * (1.5 - half_s * y * y)
    return y


def _sc_body(q_hbm, d_hbm, ii_hbm, jj_hbm, out_hbm,
             q_v, acc_v, d_v0, ii_v0, jj_v0, d_v1, ii_v1, jj_v1,
             sem0, sem1):
    cid = lax.axis_index("c")
    sid = lax.axis_index("s")
    wid = sid * NC + cid
    ebase = wid * E_W

    inv_c2 = 1.0 / (CUTOFF * CUTOFF)
    two_over_c = 2.0 / CUTOFF
    PA = -10.0 / CUTOFF_SR**3
    PB = 15.0 / CUTOFF_SR**4
    PC = -6.0 / CUTOFF_SR**5

    bufs = ((d_v0, ii_v0, jj_v0, sem0), (d_v1, ii_v1, jj_v1, sem1))

    def start(k, p):
        d_b, ii_b, jj_b, sem = bufs[p]
        base = ebase + k * CHUNK
        pltpu.async_copy(d_hbm.at[pl.ds(base, CHUNK)], d_b, sem)
        pltpu.async_copy(ii_hbm.at[pl.ds(base, CHUNK)], ii_b, sem)
        pltpu.async_copy(jj_hbm.at[pl.ds(base, CHUNK)], jj_b, sem)

    def wait(k, p):
        d_b, ii_b, jj_b, sem = bufs[p]
        base = ebase + k * CHUNK
        pltpu.make_async_copy(d_hbm.at[pl.ds(base, CHUNK)], d_b, sem).wait()
        pltpu.make_async_copy(ii_hbm.at[pl.ds(base, CHUNK)], ii_b, sem).wait()
        pltpu.make_async_copy(jj_hbm.at[pl.ds(base, CHUNK)], jj_b, sem).wait()

    with jax.named_scope("qcopy"):
        pltpu.sync_copy(q_hbm, q_v)

    zeros = jnp.zeros((LANES,), jnp.float32)

    def zero_body(v, _):
        acc_v[pl.ds(v * LANES, LANES)] = zeros
        return _

    with jax.named_scope("zero"):
        lax.fori_loop(0, N_PAD // LANES, zero_body, 0, unroll=4)

    start(0, 0)
    lane_off = lax.iota(jnp.int32, LANES) * VECS

    def process(p):
        d_b, ii_b, jj_b, _ = bufs[p]

        @plsc.parallel_loop(0, VECS, step=1, unroll=25)
        def vec_body(v):
            iv = lane_off + v
            ii = plsc.load_gather(ii_b, [iv])
            jj = plsc.load_gather(jj_b, [iv])
            d = plsc.load_gather(d_b, [iv])
            qi = plsc.load_gather(q_v, [ii])
            qj = plsc.load_gather(q_v, [jj])

            d2 = d * d
            s = d2 + 1.0
            m = d2 * s
            r = _rsqrt(m, 2)
            inv_ds = d * r
            ds = s * inv_ds
            inv_d = ds * r

            d3 = d2 * d
            p = PA + d * (PB + PC * d)
            fx = 1.0 + d3 * p
            sw_off = jnp.where(d < CUTOFF_SR, fx, 0.0)

            e_ord = inv_d + d * inv_c2 - two_over_c
            diff = (inv_ds - inv_d) + (ds - d) * inv_c2
            e = (KEHALF * qi) * qj * (e_ord + sw_off * diff)
            e = jnp.where(d <= CUTOFF, e, 0.0)
            plsc.addupdate_scatter(acc_v, [ii], e)

    def chunk_body(k, _):
        for p in range(2):
            kk = 2 * k + p

            @pl.when(kk + 1 < N_CHUNKS)
            def _start():
                start(kk + 1, 1 - p)

            wait(kk, p)
            process(p)
        return _

    with jax.named_scope("main"):
        lax.fori_loop(0, N_CHUNKS // 2, chunk_body, 0)
        wait(N_CHUNKS - 1, (N_CHUNKS - 1) % 2)
        process((N_CHUNKS - 1) % 2)

    with jax.named_scope("wout"):
        pltpu.sync_copy(acc_v, out_hbm.at[pl.ds(wid * N_PAD, N_PAD)])


def _combine_body(parts_ref, o_ref):
    o_ref[...] = jnp.sum(parts_ref[...], axis=0)


@jax.jit
def kernel(atomic_charges, distances, idx_i, idx_j):
    mesh = plsc.VectorSubcoreMesh(core_axis_name="c", subcore_axis_name="s")
    sc_fn = pl.kernel(
        _sc_body,
        out_type=jax.ShapeDtypeStruct((NW * N_PAD,), jnp.float32),
        mesh=mesh,
        compiler_params=pltpu.CompilerParams(needs_layout_passes=False),
        scratch_types=[
            pltpu.VMEM((N_NODES,), jnp.float32),
            pltpu.VMEM((N_PAD,), jnp.float32),
            pltpu.VMEM((CHUNK,), jnp.float32),
            pltpu.VMEM((CHUNK,), jnp.int32),
            pltpu.VMEM((CHUNK,), jnp.int32),
            pltpu.VMEM((CHUNK,), jnp.float32),
            pltpu.VMEM((CHUNK,), jnp.int32),
            pltpu.VMEM((CHUNK,), jnp.int32),
            pltpu.SemaphoreType.DMA,
            pltpu.SemaphoreType.DMA,
        ],
    )
    parts = sc_fn(atomic_charges, distances, idx_i, idx_j)

    combined = pl.pallas_call(
        _combine_body,
        out_shape=jax.ShapeDtypeStruct((N_PAD // 128, 128), jnp.float32),
    )(parts.reshape(NW, N_PAD // 128, 128))
    return combined.reshape(-1)[:N_NODES]


# speedup vs baseline: 665.2195x; 1.0283x over previous
import jax
import jax.numpy as jnp
from jax import lax
from jax.experimental import pallas as pl
from jax.experimental.pallas import tpu as pltpu
from jax.experimental.pallas import tpu_sc as plsc

N_NODES = 50000
N_EDGES = 3200000
CUTOFF = 10.0
CUTOFF_SR = 5.0
KEHALF = 7.199822675975274

NC = 2
NS = 16
NW = NC * NS
LANES = 16

E_W = N_EDGES // NW
CHUNK = 2000
N_CHUNKS = E_W // CHUNK
VECS = CHUNK // LANES

SLICE = 3200
N_PAD = NS * SLICE


def _rsqrt(s, iters):
    xi = plsc.bitcast(s, jnp.int32)
    yi = jnp.int32(0x5F3759DF) - lax.shift_right_logical(xi, jnp.int32(1))
    y = plsc.bitcast(yi, jnp.float32)
    half_s = 0.5 * s
    for _ in range(iters):
        y = y * (1.5 - half_s * y * y)
    return y


def _sc_body(q_hbm, d_hbm, ii_hbm, jj_hbm, out_hbm,
             q_v, acc_v, d_v0, ii_v0, jj_v0, d_v1, ii_v1, jj_v1,
             sem0, sem1):
    cid = lax.axis_index("c")
    sid = lax.axis_index("s")
    wid = sid * NC + cid
    ebase = wid * E_W

    inv_c2 = 1.0 / (CUTOFF * CUTOFF)
    two_over_c = 2.0 / CUTOFF
    PA = -10.0 / CUTOFF_SR**3
    PB = 15.0 / CUTOFF_SR**4
    PC = -6.0 / CUTOFF_SR**5

    bufs = ((d_v0, ii_v0, jj_v0, sem0), (d_v1, ii_v1, jj_v1, sem1))

    def start(k, p):
        d_b, ii_b, jj_b, sem = bufs[p]
        base = ebase + k * CHUNK
        pltpu.async_copy(d_hbm.at[pl.ds(base, CHUNK)], d_b, sem)
        pltpu.async_copy(ii_hbm.at[pl.ds(base, CHUNK)], ii_b, sem)
        pltpu.async_copy(jj_hbm.at[pl.ds(base, CHUNK)], jj_b, sem)

    def wait(k, p):
        d_b, ii_b, jj_b, sem = bufs[p]
        base = ebase + k * CHUNK
        pltpu.make_async_copy(d_hbm.at[pl.ds(base, CHUNK)], d_b, sem).wait()
        pltpu.make_async_copy(ii_hbm.at[pl.ds(base, CHUNK)], ii_b, sem).wait()
        pltpu.make_async_copy(jj_hbm.at[pl.ds(base, CHUNK)], jj_b, sem).wait()

    with jax.named_scope("qcopy"):
        pltpu.sync_copy(q_hbm, q_v)

    zeros = jnp.zeros((LANES,), jnp.float32)

    def zero_body(v, _):
        acc_v[pl.ds(v * LANES, LANES)] = zeros
        return _

    with jax.named_scope("zero"):
        lax.fori_loop(0, N_PAD // LANES, zero_body, 0, unroll=4)

    start(0, 0)
    lane_off = lax.iota(jnp.int32, LANES) * VECS

    def process(p):
        d_b, ii_b, jj_b, _ = bufs[p]

        @plsc.parallel_loop(0, VECS, step=1, unroll=25)
        def vec_body(v):
            iv = lane_off + v
            ii = plsc.load_gather(ii_b, [iv])
            jj = plsc.load_gather(jj_b, [iv])
            d = plsc.load_gather(d_b, [iv])
            qi = plsc.load_gather(q_v, [ii])
            qj = plsc.load_gather(q_v, [jj])

            d2 = d * d
            s = d2 + 1.0
            m = d2 * s
            r = _rsqrt(m, 2)
            inv_ds = d * r
            ds = s * inv_ds
            inv_d = ds * r

            d3 = d2 * d
            p = PA + d * (PB + PC * d)
            fx = 1.0 + d3 * p
            sw_off = jnp.where(d < CUTOFF_SR, fx, 0.0)

            e_ord = inv_d + d * inv_c2 - two_over_c
            diff = (inv_ds - inv_d) + (ds - d) * inv_c2
            e = (KEHALF * qi) * qj * (e_ord + sw_off * diff)
            e = jnp.where(d <= CUTOFF, e, 0.0)
            plsc.addupdate_scatter(acc_v, [ii], e)

    def chunk_body(k, _):
        for p in range(2):
            kk = 2 * k + p

            @pl.when(kk + 1 < N_CHUNKS)
            def _start():
                start(kk + 1, 1 - p)

            wait(kk, p)
            process(p)
        return _

    with jax.named_scope("main"):
        lax.fori_loop(0, N_CHUNKS // 2, chunk_body, 0)
        if N_CHUNKS % 2:
            wait(N_CHUNKS - 1, (N_CHUNKS - 1) % 2)
            process((N_CHUNKS - 1) % 2)

    with jax.named_scope("wout"):
        pltpu.sync_copy(acc_v, out_hbm.at[pl.ds(wid * N_PAD, N_PAD)])


def _combine_body(parts_ref, o_ref):
    o_ref[...] = jnp.sum(parts_ref[...], axis=0)


@jax.jit
def kernel(atomic_charges, distances, idx_i, idx_j):
    mesh = plsc.VectorSubcoreMesh(core_axis_name="c", subcore_axis_name="s")
    sc_fn = pl.kernel(
        _sc_body,
        out_type=jax.ShapeDtypeStruct((NW * N_PAD,), jnp.float32),
        mesh=mesh,
        compiler_params=pltpu.CompilerParams(needs_layout_passes=False),
        scratch_types=[
            pltpu.VMEM((N_NODES,), jnp.float32),
            pltpu.VMEM((N_PAD,), jnp.float32),
            pltpu.VMEM((CHUNK,), jnp.float32),
            pltpu.VMEM((CHUNK,), jnp.int32),
            pltpu.VMEM((CHUNK,), jnp.int32),
            pltpu.VMEM((CHUNK,), jnp.float32),
            pltpu.VMEM((CHUNK,), jnp.int32),
            pltpu.VMEM((CHUNK,), jnp.int32),
            pltpu.SemaphoreType.DMA,
            pltpu.SemaphoreType.DMA,
        ],
    )
    parts = sc_fn(atomic_charges, distances, idx_i, idx_j)

    combined = pl.pallas_call(
        _combine_body,
        out_shape=jax.ShapeDtypeStruct((N_PAD // 128, 128), jnp.float32),
    )(parts.reshape(NW, N_PAD // 128, 128))
    return combined.reshape(-1)[:N_NODES]
